# Initial kernel scaffold; baseline (speedup 1.0000x reference)
import jax
import jax.numpy as jnp
from jax.experimental import pallas as pl


def kernel(x, edge_index, edge_attr, We_w, We_b, root, conv_bias, Wp, bp):
    raise NotImplementedError("write your pallas kernel here")


# speedup vs baseline: 2.9449x; 2.9449x over previous
import functools

import jax
import jax.numpy as jnp
from jax import lax
from jax.experimental import pallas as pl
from jax.experimental.pallas import tpu as pltpu
from jax.experimental.pallas import tpu_sc as plsc

D_IN = 128
D_EDGE = 16
D_OUT = 8
KX = D_EDGE + 1
GW = KX * D_OUT + D_OUT
CHUNK = 128
NW = 32


def _node_matmul(x, wbig, n_blocks, blk):
    def body(x_ref, w_ref, o_ref):
        o_ref[:, :] = jnp.dot(x_ref[:, :], w_ref[:, :],
                              preferred_element_type=jnp.float32)

    n = x.shape[0]
    return pl.pallas_call(
        body,
        grid=(n_blocks,),
        in_specs=[pl.BlockSpec((blk, D_IN), lambda i: (i, 0)),
                  pl.BlockSpec((D_IN, GW), lambda i: (0, 0))],
        out_specs=pl.BlockSpec((blk, GW), lambda i: (i, 0)),
        out_shape=jax.ShapeDtypeStruct((n, GW), jnp.float32),
    )(x, wbig)


def _make_sc_edges(n_acc, rows_per_tile, n_chunks, epw):
    mesh = plsc.VectorSubcoreMesh(core_axis_name="c", subcore_axis_name="s")

    @functools.partial(
        pl.kernel,
        mesh=mesh,
        compiler_params=pltpu.CompilerParams(use_tc_tiling_on_sc=False,
                                             needs_layout_passes=False),
        out_type=jax.ShapeDtypeStruct((2, n_acc, D_OUT), jnp.float32),
        scratch_types=[
            pltpu.VMEM((CHUNK,), jnp.int32),
            pltpu.VMEM((CHUNK,), jnp.int32),
            pltpu.VMEM((KX, CHUNK), jnp.float32),
            pltpu.VMEM((CHUNK, GW), jnp.float32),
            pltpu.VMEM((CHUNK, D_OUT), jnp.float32),
            pltpu.VMEM_SHARED((n_acc, D_OUT), jnp.float32),
        ],
    )
    def sc_edges(g_hbm, src_hbm, dst_hbm, attr_hbm, zeros_hbm, out_hbm,
                 src_v, dst_v, attr_v, rows_v, msg_v, acc_s):
        c = lax.axis_index("c")
        s = lax.axis_index("s")
        wid = s * 2 + c

        row0 = s * rows_per_tile
        pltpu.sync_copy(zeros_hbm.at[pl.ds(row0, rows_per_tile)],
                        acc_s.at[pl.ds(row0, rows_per_tile)])
        plsc.subcore_barrier()

        ebase = wid * epw

        def chunk_body(ci, carry):
            base = ebase + ci * CHUNK
            pltpu.sync_copy(src_hbm.at[pl.ds(base, CHUNK)], src_v)
            pltpu.sync_copy(dst_hbm.at[pl.ds(base, CHUNK)], dst_v)
            pltpu.sync_copy(attr_hbm.at[wid * n_chunks + ci], attr_v)
            pltpu.sync_copy(g_hbm.at[src_v], rows_v)

            def group_body(g, carry2):
                eidx = lax.iota(jnp.int32, 16) + g * 16
                acc = [jnp.zeros((16,), jnp.float32) for _ in range(D_OUT)]
                for k in range(KX):
                    a_k = attr_v[k, pl.ds(g * 16, 16)]
                    for o in range(D_OUT):
                        col = jnp.full((16,), k * D_OUT + o, jnp.int32)
                        vals = plsc.load_gather(rows_v, [eidx, col])
                        acc[o] = acc[o] + a_k * vals
                for o in range(D_OUT):
                    colo = jnp.full((16,), o, jnp.int32)
                    plsc.store_scatter(msg_v, [eidx, colo], acc[o])
                return carry2

            lax.fori_loop(0, CHUNK // 16, group_body, 0)
            pltpu.sync_copy(msg_v, acc_s.at[dst_v], add=True)
            return carry

        lax.fori_loop(0, n_chunks, chunk_body, 0)

        plsc.subcore_barrier()
        pltpu.sync_copy(acc_s.at[pl.ds(row0, rows_per_tile)],
                        out_hbm.at[c, pl.ds(row0, rows_per_tile)])

    return sc_edges


def _final(part, g, conv_bias, wp, bp, n_blocks, blk):
    def body(p_ref, g_ref, cb_ref, wp_ref, bp_ref, o_ref):
        aggr = p_ref[0] + p_ref[1]
        pre = aggr + g_ref[:, KX * D_OUT:] + cb_ref[:, :]
        h = jnp.maximum(pre, 0.0)
        o_ref[:, :] = (jnp.dot(h, wp_ref[:, :],
                               preferred_element_type=jnp.float32)
                       + bp_ref[:, :])

    n = g.shape[0]
    return pl.pallas_call(
        body,
        grid=(n_blocks,),
        in_specs=[
            pl.BlockSpec((2, blk, D_OUT), lambda i: (0, i, 0)),
            pl.BlockSpec((blk, GW), lambda i: (i, 0)),
            pl.BlockSpec((1, D_OUT), lambda i: (0, 0)),
            pl.BlockSpec((D_OUT, D_OUT), lambda i: (0, 0)),
            pl.BlockSpec((1, D_OUT), lambda i: (0, 0)),
        ],
        out_specs=pl.BlockSpec((blk, D_OUT), lambda i: (i, 0)),
        out_shape=jax.ShapeDtypeStruct((n, D_OUT), jnp.float32),
    )(part, g, conv_bias, wp, bp)


def kernel(x, edge_index, edge_attr, We_w, We_b, root, conv_bias, Wp, bp):
    x = x.astype(jnp.float32)
    n = x.shape[0]
    e = edge_attr.shape[0]
    src = edge_index[0].astype(jnp.int32)
    dst = edge_index[1].astype(jnp.int32)

    epw = -(-e // (NW * CHUNK)) * CHUNK
    e_pad = NW * epw
    n_chunks = epw // CHUNK
    rows_per_tile = 8 * (-(-(n + 1) // (16 * 8)))
    n_acc = 16 * rows_per_tile

    wm = We_w.reshape(D_EDGE, D_IN, D_OUT).transpose(1, 0, 2)
    wm = wm.reshape(D_IN, D_EDGE * D_OUT)
    bm = We_b.reshape(D_IN, D_OUT)
    wbig = jnp.concatenate([wm, bm, root], axis=1)

    g = _node_matmul(x, wbig, 10, n // 10)

    pad = e_pad - e
    src_p = jnp.concatenate([src, jnp.zeros((pad,), jnp.int32)])
    dst_p = jnp.concatenate([dst, jnp.full((pad,), n, jnp.int32)])
    attr_ext = jnp.concatenate(
        [edge_attr.astype(jnp.float32), jnp.ones((e, 1), jnp.float32)], axis=1)
    attr_ext = jnp.concatenate(
        [attr_ext, jnp.zeros((pad, KX), jnp.float32)], axis=0)
    attr_blk = attr_ext.reshape(e_pad // CHUNK, CHUNK, KX).transpose(0, 2, 1)

    zeros = jnp.zeros((n_acc, D_OUT), jnp.float32)
    sc = _make_sc_edges(n_acc, rows_per_tile, n_chunks, epw)
    part = sc(g, src_p, dst_p, attr_blk, zeros)

    part = part[:, :n, :]
    return _final(part, g, conv_bias.reshape(1, D_OUT), Wp,
                  bp.reshape(1, D_OUT), 10, n // 10)


# speedup vs baseline: 3.5166x; 1.1941x over previous
import functools

import jax
import jax.numpy as jnp
from jax import lax
from jax.experimental import pallas as pl
from jax.experimental.pallas import tpu as pltpu
from jax.experimental.pallas import tpu_sc as plsc

D_IN = 128
D_EDGE = 16
D_OUT = 8
KX = D_EDGE + 1
GW = KX * D_OUT + D_OUT
CHUNK = 128
NW = 32


def _node_matmul(x, wbig, n_blocks, blk):
    def body(x_ref, w_ref, o_ref):
        o_ref[:, :] = jnp.dot(x_ref[:, :], w_ref[:, :],
                              preferred_element_type=jnp.float32)

    n = x.shape[0]
    return pl.pallas_call(
        body,
        grid=(n_blocks,),
        in_specs=[pl.BlockSpec((blk, D_IN), lambda i: (i, 0)),
                  pl.BlockSpec((D_IN, GW), lambda i: (0, 0))],
        out_specs=pl.BlockSpec((blk, GW), lambda i: (i, 0)),
        out_shape=jax.ShapeDtypeStruct((n, GW), jnp.float32),
    )(x, wbig)


def _make_sc_edges(n_acc, rows_per_tile, n_chunks, ept, last_thresh):
    mesh = plsc.VectorSubcoreMesh(core_axis_name="c", subcore_axis_name="s")

    @functools.partial(
        pl.kernel,
        mesh=mesh,
        compiler_params=pltpu.CompilerParams(use_tc_tiling_on_sc=False,
                                             needs_layout_passes=False),
        out_type=jax.ShapeDtypeStruct((2, n_acc, D_OUT), jnp.float32),
        scratch_types=[
            pltpu.VMEM((CHUNK,), jnp.int32),
            pltpu.VMEM((CHUNK,), jnp.int32),
            pltpu.VMEM((CHUNK, D_EDGE), jnp.float32),
            pltpu.VMEM((CHUNK, GW), jnp.float32),
            pltpu.VMEM((CHUNK, D_OUT), jnp.float32),
            pltpu.VMEM_SHARED((n_acc, D_OUT), jnp.float32),
        ],
    )
    def sc_edges(g_hbm, src_hbm, dst_hbm, attr_hbm, zeros_hbm, out_hbm,
                 src_v, dst_v, attr_v, rows_v, msg_v, acc_s):
        c = lax.axis_index("c")
        s = lax.axis_index("s")
        wid = s * 2 + c

        row0 = s * rows_per_tile
        pltpu.sync_copy(zeros_hbm.at[pl.ds(row0, rows_per_tile)],
                        acc_s.at[pl.ds(row0, rows_per_tile)])
        plsc.subcore_barrier()

        ebase = wid * ept

        def chunk_body(ci, carry):
            is_last = ci == n_chunks - 1
            off = jnp.where(is_last, ept - CHUNK, ci * CHUNK)
            thresh = jnp.where(is_last, last_thresh, 0)
            base = ebase + off
            pltpu.sync_copy(src_hbm.at[pl.ds(base, CHUNK)], src_v)
            pltpu.sync_copy(dst_hbm.at[pl.ds(base, CHUNK)], dst_v)
            pltpu.sync_copy(attr_hbm.at[pl.ds(base, CHUNK)], attr_v)
            pltpu.sync_copy(g_hbm.at[src_v], rows_v)

            def group_body(g, carry2):
                eidx = lax.iota(jnp.int32, 16) + g * 16
                acc = [jnp.zeros((16,), jnp.float32) for _ in range(D_OUT)]
                for k in range(D_EDGE):
                    a_k = plsc.load_gather(
                        attr_v, [eidx, jnp.full((16,), k, jnp.int32)])
                    for o in range(D_OUT):
                        col = jnp.full((16,), k * D_OUT + o, jnp.int32)
                        vals = plsc.load_gather(rows_v, [eidx, col])
                        acc[o] = acc[o] + a_k * vals
                keep = eidx >= thresh
                for o in range(D_OUT):
                    colb = jnp.full((16,), D_EDGE * D_OUT + o, jnp.int32)
                    bias = plsc.load_gather(rows_v, [eidx, colb])
                    val = jnp.where(keep, acc[o] + bias, 0.0)
                    colo = jnp.full((16,), o, jnp.int32)
                    plsc.store_scatter(msg_v, [eidx, colo], val)
                return carry2

            lax.fori_loop(0, CHUNK // 16, group_body, 0)
            pltpu.sync_copy(msg_v, acc_s.at[dst_v], add=True)
            return carry

        lax.fori_loop(0, n_chunks, chunk_body, 0)

        plsc.subcore_barrier()
        pltpu.sync_copy(acc_s.at[pl.ds(row0, rows_per_tile)],
                        out_hbm.at[c, pl.ds(row0, rows_per_tile)])

    return sc_edges


def _final(part, g, conv_bias, wp, bp, n_blocks, blk):
    def body(p_ref, g_ref, cb_ref, wp_ref, bp_ref, o_ref):
        aggr = p_ref[0] + p_ref[1]
        pre = aggr + g_ref[:, KX * D_OUT:] + cb_ref[:, :]
        h = jnp.maximum(pre, 0.0)
        o_ref[:, :] = (jnp.dot(h, wp_ref[:, :],
                               preferred_element_type=jnp.float32)
                       + bp_ref[:, :])

    n = g.shape[0]
    return pl.pallas_call(
        body,
        grid=(n_blocks,),
        in_specs=[
            pl.BlockSpec((2, blk, D_OUT), lambda i: (0, i, 0)),
            pl.BlockSpec((blk, GW), lambda i: (i, 0)),
            pl.BlockSpec((1, D_OUT), lambda i: (0, 0)),
            pl.BlockSpec((D_OUT, D_OUT), lambda i: (0, 0)),
            pl.BlockSpec((1, D_OUT), lambda i: (0, 0)),
        ],
        out_specs=pl.BlockSpec((blk, D_OUT), lambda i: (i, 0)),
        out_shape=jax.ShapeDtypeStruct((n, D_OUT), jnp.float32),
    )(part, g, conv_bias, wp, bp)


def kernel(x, edge_index, edge_attr, We_w, We_b, root, conv_bias, Wp, bp):
    x = x.astype(jnp.float32)
    n = x.shape[0]
    e = edge_attr.shape[0]
    src = edge_index[0].astype(jnp.int32)
    dst = edge_index[1].astype(jnp.int32)

    assert e % NW == 0 and (e // NW) % 8 == 0 and e // NW >= CHUNK
    ept = e // NW
    n_chunks = -(-ept // CHUNK)
    last_thresh = (CHUNK - ept % CHUNK) % CHUNK
    rows_per_tile = 8 * (-(-(n + 1) // (16 * 8)))
    n_acc = 16 * rows_per_tile

    wm = We_w.reshape(D_EDGE, D_IN, D_OUT).transpose(1, 0, 2)
    wm = wm.reshape(D_IN, D_EDGE * D_OUT)
    bm = We_b.reshape(D_IN, D_OUT)
    wbig = jnp.concatenate([wm, bm, root], axis=1)

    g = _node_matmul(x, wbig, 10, n // 10)

    zeros = jnp.zeros((n_acc, D_OUT), jnp.float32)
    sc = _make_sc_edges(n_acc, rows_per_tile, n_chunks, ept, last_thresh)
    part = sc(g, src, dst, edge_attr.astype(jnp.float32), zeros)

    part = part[:, :n, :]
    return _final(part, g, conv_bias.reshape(1, D_OUT), Wp,
                  bp.reshape(1, D_OUT), 10, n // 10)


# speedup vs baseline: 4.2042x; 1.1955x over previous
import functools

import jax
import jax.numpy as jnp
from jax import lax
from jax.experimental import pallas as pl
from jax.experimental.pallas import tpu as pltpu
from jax.experimental.pallas import tpu_sc as plsc

D_IN = 128
D_EDGE = 16
D_OUT = 8
KX = D_EDGE + 1
GW = KX * D_OUT + D_OUT
CHUNK = 128
NW = 32


def _node_matmul(x, wbig, n_blocks, blk):
    def body(x_ref, w_ref, o_ref):
        o_ref[:, :] = jnp.dot(x_ref[:, :], w_ref[:, :],
                              preferred_element_type=jnp.float32)

    n = x.shape[0]
    return pl.pallas_call(
        body,
        grid=(n_blocks,),
        in_specs=[pl.BlockSpec((blk, D_IN), lambda i: (i, 0)),
                  pl.BlockSpec((D_IN, GW), lambda i: (0, 0))],
        out_specs=pl.BlockSpec((blk, GW), lambda i: (i, 0)),
        out_shape=jax.ShapeDtypeStruct((n, GW), jnp.float32),
    )(x, wbig)


def _make_sc_edges(n_acc, rows_per_tile, n_chunks, ept, last_thresh):
    mesh = plsc.VectorSubcoreMesh(core_axis_name="c", subcore_axis_name="s")
    assert n_chunks >= 3
    nb, tail = divmod(n_chunks, 3)

    scratch = (
        [pltpu.VMEM((CHUNK,), jnp.int32)] * 3
        + [pltpu.VMEM((CHUNK,), jnp.int32)] * 3
        + [pltpu.VMEM((CHUNK, D_EDGE), jnp.float32)] * 3
        + [pltpu.VMEM((CHUNK, GW), jnp.float32)] * 3
        + [pltpu.VMEM((CHUNK, D_OUT), jnp.float32)] * 3
        + [pltpu.VMEM_SHARED((n_acc, D_OUT), jnp.float32)]
        + [pltpu.SemaphoreType.DMA] * 9
    )

    @functools.partial(
        pl.kernel,
        mesh=mesh,
        compiler_params=pltpu.CompilerParams(use_tc_tiling_on_sc=False,
                                             needs_layout_passes=False),
        out_type=jax.ShapeDtypeStruct((2, n_acc, D_OUT), jnp.float32),
        scratch_types=scratch,
    )
    def sc_edges(g_hbm, src_hbm, dst_hbm, attr_hbm, zeros_hbm, out_hbm,
                 *bufs):
        src_b = bufs[0:3]
        dst_b = bufs[3:6]
        attr_b = bufs[6:9]
        rows_b = bufs[9:12]
        msg_b = bufs[12:15]
        acc_s = bufs[15]
        se_idx = bufs[16:19]
        se_rows = bufs[19:22]
        se_sc = bufs[22:25]

        c = lax.axis_index("c")
        s = lax.axis_index("s")
        wid = s * 2 + c
        ebase = wid * ept

        def run(cond, fn):
            if isinstance(cond, bool):
                if cond:
                    fn()
            else:
                pl.when(cond)(fn)

        def chunk_off(ci):
            if isinstance(ci, int):
                return ept - CHUNK if ci == n_chunks - 1 else ci * CHUNK
            return jnp.where(ci == n_chunks - 1, ept - CHUNK, ci * CHUNK)

        def issue_idx(ci, b):
            base = ebase + chunk_off(ci)
            pltpu.async_copy(src_hbm.at[pl.ds(base, CHUNK)], src_b[b],
                             se_idx[b])
            pltpu.async_copy(dst_hbm.at[pl.ds(base, CHUNK)], dst_b[b],
                             se_idx[b])
            pltpu.async_copy(attr_hbm.at[pl.ds(base, CHUNK)], attr_b[b],
                             se_idx[b])

        def wait_idx(b):
            pltpu.make_async_copy(src_hbm.at[pl.ds(0, CHUNK)], src_b[b],
                                  se_idx[b]).wait()
            pltpu.make_async_copy(dst_hbm.at[pl.ds(0, CHUNK)], dst_b[b],
                                  se_idx[b]).wait()
            pltpu.make_async_copy(attr_hbm.at[pl.ds(0, CHUNK)], attr_b[b],
                                  se_idx[b]).wait()

        def issue_rows(b):
            pltpu.async_copy(g_hbm.at[src_b[b]], rows_b[b], se_rows[b])

        def wait_rows(b):
            pltpu.make_async_copy(g_hbm.at[src_b[b]], rows_b[b],
                                  se_rows[b]).wait()

        def issue_scatter(b):
            pltpu.async_copy(msg_b[b], acc_s.at[dst_b[b]], se_sc[b],
                             add=True)

        def wait_scatter(b):
            pltpu.make_async_copy(msg_b[b], acc_s.at[dst_b[b]],
                                  se_sc[b]).wait()

        def compute(ci, b):
            thresh = jnp.where(ci == n_chunks - 1, last_thresh, 0)
            rows_v = rows_b[b]
            attr_v = attr_b[b]
            msg_v = msg_b[b]

            def group_body(g, carry2):
                eidx = lax.iota(jnp.int32, 16) + g * 16
                acc = [jnp.zeros((16,), jnp.float32) for _ in range(D_OUT)]
                for k in range(D_EDGE):
                    a_k = plsc.load_gather(
                        attr_v, [eidx, jnp.full((16,), k, jnp.int32)])
                    for o in range(D_OUT):
                        col = jnp.full((16,), k * D_OUT + o, jnp.int32)
                        vals = plsc.load_gather(rows_v, [eidx, col])
                        acc[o] = acc[o] + a_k * vals
                keep = eidx >= thresh
                for o in range(D_OUT):
                    colb = jnp.full((16,), D_EDGE * D_OUT + o, jnp.int32)
                    bias = plsc.load_gather(rows_v, [eidx, colb])
                    val = jnp.where(keep, acc[o] + bias, 0.0)
                    colo = jnp.full((16,), o, jnp.int32)
                    plsc.store_scatter(msg_v, [eidx, colo], val)
                return carry2

            lax.fori_loop(0, CHUNK // 16, group_body, 0)

        def chunk_step(ci, b):
            wait_rows(b)
            b1 = (b + 1) % 3
            b2 = (b + 2) % 3
            def prefetch_rows():
                wait_idx(b1)
                issue_rows(b1)

            run(ci + 1 < n_chunks, prefetch_rows)
            compute(ci, b)

            def prefetch_idx():
                run(ci >= 1, lambda: wait_scatter(b2))
                issue_idx(ci + 2, b2)

            if isinstance(ci, int):
                if ci + 2 < n_chunks:
                    if ci >= 1:
                        wait_scatter(b2)
                    issue_idx(ci + 2, b2)
            else:
                run(ci + 2 < n_chunks, prefetch_idx)
            issue_scatter(b)

        issue_idx(0, 0)
        issue_idx(1, 1)
        row0 = s * rows_per_tile
        pltpu.sync_copy(zeros_hbm.at[pl.ds(row0, rows_per_tile)],
                        acc_s.at[pl.ds(row0, rows_per_tile)])
        plsc.subcore_barrier()
        wait_idx(0)
        issue_rows(0)

        def block_body(blk, carry):
            ci0 = blk * 3
            chunk_step(ci0, 0)
            chunk_step(ci0 + 1, 1)
            chunk_step(ci0 + 2, 2)
            return carry

        lax.fori_loop(0, nb, block_body, 0)
        for t in range(tail):
            chunk_step(nb * 3 + t, t)

        for j in range(n_chunks - 3, n_chunks):
            wait_scatter(j % 3)

        plsc.subcore_barrier()
        pltpu.sync_copy(acc_s.at[pl.ds(row0, rows_per_tile)],
                        out_hbm.at[c, pl.ds(row0, rows_per_tile)])

    return sc_edges


def _final(part, g, conv_bias, wp, bp, n_blocks, blk):
    def body(p_ref, g_ref, cb_ref, wp_ref, bp_ref, o_ref):
        aggr = p_ref[0] + p_ref[1]
        pre = aggr + g_ref[:, KX * D_OUT:] + cb_ref[:, :]
        h = jnp.maximum(pre, 0.0)
        o_ref[:, :] = (jnp.dot(h, wp_ref[:, :],
                               preferred_element_type=jnp.float32)
                       + bp_ref[:, :])

    n = g.shape[0]
    return pl.pallas_call(
        body,
        grid=(n_blocks,),
        in_specs=[
            pl.BlockSpec((2, blk, D_OUT), lambda i: (0, i, 0)),
            pl.BlockSpec((blk, GW), lambda i: (i, 0)),
            pl.BlockSpec((1, D_OUT), lambda i: (0, 0)),
            pl.BlockSpec((D_OUT, D_OUT), lambda i: (0, 0)),
            pl.BlockSpec((1, D_OUT), lambda i: (0, 0)),
        ],
        out_specs=pl.BlockSpec((blk, D_OUT), lambda i: (i, 0)),
        out_shape=jax.ShapeDtypeStruct((n, D_OUT), jnp.float32),
    )(part, g, conv_bias, wp, bp)


def kernel(x, edge_index, edge_attr, We_w, We_b, root, conv_bias, Wp, bp):
    x = x.astype(jnp.float32)
    n = x.shape[0]
    e = edge_attr.shape[0]
    src = edge_index[0].astype(jnp.int32)
    dst = edge_index[1].astype(jnp.int32)

    assert e % NW == 0 and (e // NW) % 8 == 0 and e // NW >= CHUNK
    ept = e // NW
    n_chunks = -(-ept // CHUNK)
    last_thresh = (CHUNK - ept % CHUNK) % CHUNK
    rows_per_tile = 8 * (-(-(n + 1) // (16 * 8)))
    n_acc = 16 * rows_per_tile

    wm = We_w.reshape(D_EDGE, D_IN, D_OUT).transpose(1, 0, 2)
    wm = wm.reshape(D_IN, D_EDGE * D_OUT)
    bm = We_b.reshape(D_IN, D_OUT)
    wbig = jnp.concatenate([wm, bm, root], axis=1)

    g = _node_matmul(x, wbig, 10, n // 10)

    zeros = jnp.zeros((n_acc, D_OUT), jnp.float32)
    sc = _make_sc_edges(n_acc, rows_per_tile, n_chunks, ept, last_thresh)
    part = sc(g, src, dst, edge_attr.astype(jnp.float32), zeros)

    part = part[:, :n, :]
    return _final(part, g, conv_bias.reshape(1, D_OUT), Wp,
                  bp.reshape(1, D_OUT), 10, n // 10)


# speedup vs baseline: 4.8887x; 1.1628x over previous
import functools

import jax
import jax.numpy as jnp
from jax import lax
from jax.experimental import pallas as pl
from jax.experimental.pallas import tpu as pltpu
from jax.experimental.pallas import tpu_sc as plsc

D_IN = 128
D_EDGE = 16
D_OUT = 8
KX = D_EDGE + 1
GW = KX * D_OUT + D_OUT
GP = 152
CHUNK = 128
NW = 32


def _node_matmul(x, wbig, n_blocks, blk):
    def body(x_ref, w_ref, o_ref):
        o_ref[:, :] = jnp.dot(x_ref[:, :], w_ref[:, :],
                              preferred_element_type=jnp.float32)

    n = x.shape[0]
    return pl.pallas_call(
        body,
        grid=(n_blocks,),
        in_specs=[pl.BlockSpec((blk, D_IN), lambda i: (i, 0)),
                  pl.BlockSpec((D_IN, GP), lambda i: (0, 0))],
        out_specs=pl.BlockSpec((blk, GP), lambda i: (i, 0)),
        out_shape=jax.ShapeDtypeStruct((n, GP), jnp.float32),
    )(x, wbig)


def _make_sc_edges(n_acc, rows_per_tile, n_chunks, ept, last_thresh):
    mesh = plsc.VectorSubcoreMesh(core_axis_name="c", subcore_axis_name="s")
    assert n_chunks >= 3
    nb, tail = divmod(n_chunks, 3)

    scratch = (
        [pltpu.VMEM((CHUNK,), jnp.int32)] * 3
        + [pltpu.VMEM((CHUNK,), jnp.int32)] * 3
        + [pltpu.VMEM((CHUNK, D_EDGE + 1), jnp.float32)] * 3
        + [pltpu.VMEM((CHUNK, GP), jnp.float32)] * 3
        + [pltpu.VMEM((CHUNK, D_OUT), jnp.float32)] * 3
        + [pltpu.VMEM_SHARED((n_acc, D_OUT), jnp.float32)]
        + [pltpu.SemaphoreType.DMA] * 9
    )

    @functools.partial(
        pl.kernel,
        mesh=mesh,
        compiler_params=pltpu.CompilerParams(use_tc_tiling_on_sc=False,
                                             needs_layout_passes=False),
        out_type=jax.ShapeDtypeStruct((2, n_acc, D_OUT), jnp.float32),
        scratch_types=scratch,
    )
    def sc_edges(g_hbm, src_hbm, dst_hbm, attr_hbm, zeros_hbm, out_hbm,
                 *bufs):
        src_b = bufs[0:3]
        dst_b = bufs[3:6]
        attr_b = bufs[6:9]
        rows_b = bufs[9:12]
        msg_b = bufs[12:15]
        acc_s = bufs[15]
        se_idx = bufs[16:19]
        se_rows = bufs[19:22]
        se_sc = bufs[22:25]

        c = lax.axis_index("c")
        s = lax.axis_index("s")
        wid = s * 2 + c
        ebase = wid * ept

        def run(cond, fn):
            if isinstance(cond, bool):
                if cond:
                    fn()
            else:
                pl.when(cond)(fn)

        def chunk_off(ci):
            if isinstance(ci, int):
                return ept - CHUNK if ci == n_chunks - 1 else ci * CHUNK
            return jnp.where(ci == n_chunks - 1, ept - CHUNK, ci * CHUNK)

        def issue_idx(ci, b):
            base = ebase + chunk_off(ci)
            pltpu.async_copy(src_hbm.at[pl.ds(base, CHUNK)], src_b[b],
                             se_idx[b])
            pltpu.async_copy(dst_hbm.at[pl.ds(base, CHUNK)], dst_b[b],
                             se_idx[b])
            pltpu.async_copy(attr_hbm.at[pl.ds(base, CHUNK)],
                             attr_b[b].at[:, pl.ds(0, D_EDGE)], se_idx[b])

        def wait_idx(b):
            pltpu.make_async_copy(src_hbm.at[pl.ds(0, CHUNK)], src_b[b],
                                  se_idx[b]).wait()
            pltpu.make_async_copy(dst_hbm.at[pl.ds(0, CHUNK)], dst_b[b],
                                  se_idx[b]).wait()
            pltpu.make_async_copy(attr_hbm.at[pl.ds(0, CHUNK)],
                                  attr_b[b].at[:, pl.ds(0, D_EDGE)],
                                  se_idx[b]).wait()

        def issue_rows(b):
            pltpu.async_copy(g_hbm.at[src_b[b]], rows_b[b], se_rows[b])

        def wait_rows(b):
            pltpu.make_async_copy(g_hbm.at[src_b[b]], rows_b[b],
                                  se_rows[b]).wait()

        def issue_scatter(b):
            pltpu.async_copy(msg_b[b], acc_s.at[dst_b[b]], se_sc[b],
                             add=True)

        def wait_scatter(b):
            pltpu.make_async_copy(msg_b[b], acc_s.at[dst_b[b]],
                                  se_sc[b]).wait()

        def compute(ci, b):
            thresh = jnp.where(ci == n_chunks - 1, last_thresh, 0)
            rows_v = rows_b[b]
            attr_v = attr_b[b]
            msg_v = msg_b[b]

            def group_body(g, carry2):
                eidx = lax.iota(jnp.int32, 16) + g * 16
                acc = [jnp.zeros((16,), jnp.float32) for _ in range(D_OUT)]
                for k in range(D_EDGE):
                    a_k = plsc.load_gather(
                        attr_v, [eidx, jnp.full((16,), k, jnp.int32)])
                    for o in range(D_OUT):
                        col = jnp.full((16,), k * D_OUT + o, jnp.int32)
                        vals = plsc.load_gather(rows_v, [eidx, col])
                        acc[o] = acc[o] + a_k * vals
                keep = eidx >= thresh
                for o in range(D_OUT):
                    colb = jnp.full((16,), D_EDGE * D_OUT + o, jnp.int32)
                    bias = plsc.load_gather(rows_v, [eidx, colb])
                    val = jnp.where(keep, acc[o] + bias, 0.0)
                    colo = jnp.full((16,), o, jnp.int32)
                    plsc.store_scatter(msg_v, [eidx, colo], val)
                return carry2

            lax.fori_loop(0, CHUNK // 16, group_body, 0)

        def chunk_step(ci, b):
            wait_rows(b)
            b1 = (b + 1) % 3
            b2 = (b + 2) % 3
            def prefetch_rows():
                wait_idx(b1)
                issue_rows(b1)

            run(ci + 1 < n_chunks, prefetch_rows)
            compute(ci, b)

            def prefetch_idx():
                run(ci >= 1, lambda: wait_scatter(b2))
                issue_idx(ci + 2, b2)

            if isinstance(ci, int):
                if ci + 2 < n_chunks:
                    if ci >= 1:
                        wait_scatter(b2)
                    issue_idx(ci + 2, b2)
            else:
                run(ci + 2 < n_chunks, prefetch_idx)
            issue_scatter(b)

        issue_idx(0, 0)
        issue_idx(1, 1)
        row0 = s * rows_per_tile
        pltpu.sync_copy(zeros_hbm.at[pl.ds(row0, rows_per_tile)],
                        acc_s.at[pl.ds(row0, rows_per_tile)])
        plsc.subcore_barrier()
        wait_idx(0)
        issue_rows(0)

        def block_body(blk, carry):
            ci0 = blk * 3
            chunk_step(ci0, 0)
            chunk_step(ci0 + 1, 1)
            chunk_step(ci0 + 2, 2)
            return carry

        lax.fori_loop(0, nb, block_body, 0)
        for t in range(tail):
            chunk_step(nb * 3 + t, t)

        for j in range(n_chunks - 3, n_chunks):
            wait_scatter(j % 3)

        plsc.subcore_barrier()
        pltpu.sync_copy(acc_s.at[pl.ds(row0, rows_per_tile)],
                        out_hbm.at[c, pl.ds(row0, rows_per_tile)])

    return sc_edges


def _final(part, g, conv_bias, wp, bp, n_blocks, blk):
    def body(p_ref, g_ref, cb_ref, wp_ref, bp_ref, o_ref):
        aggr = p_ref[0] + p_ref[1]
        pre = aggr + g_ref[:, KX * D_OUT:GW] + cb_ref[:, :]
        h = jnp.maximum(pre, 0.0)
        o_ref[:, :] = (jnp.dot(h, wp_ref[:, :],
                               preferred_element_type=jnp.float32)
                       + bp_ref[:, :])

    n = g.shape[0]
    return pl.pallas_call(
        body,
        grid=(n_blocks,),
        in_specs=[
            pl.BlockSpec((2, blk, D_OUT), lambda i: (0, i, 0)),
            pl.BlockSpec((blk, GP), lambda i: (i, 0)),
            pl.BlockSpec((1, D_OUT), lambda i: (0, 0)),
            pl.BlockSpec((D_OUT, D_OUT), lambda i: (0, 0)),
            pl.BlockSpec((1, D_OUT), lambda i: (0, 0)),
        ],
        out_specs=pl.BlockSpec((blk, D_OUT), lambda i: (i, 0)),
        out_shape=jax.ShapeDtypeStruct((n, D_OUT), jnp.float32),
    )(part, g, conv_bias, wp, bp)


def kernel(x, edge_index, edge_attr, We_w, We_b, root, conv_bias, Wp, bp):
    x = x.astype(jnp.float32)
    n = x.shape[0]
    e = edge_attr.shape[0]
    src = edge_index[0].astype(jnp.int32)
    dst = edge_index[1].astype(jnp.int32)

    assert e % NW == 0 and (e // NW) % 8 == 0 and e // NW >= CHUNK
    ept = e // NW
    n_chunks = -(-ept // CHUNK)
    last_thresh = (CHUNK - ept % CHUNK) % CHUNK
    rows_per_tile = 8 * (-(-(n + 1) // (16 * 8)))
    n_acc = 16 * rows_per_tile

    wm = We_w.reshape(D_EDGE, D_IN, D_OUT).transpose(1, 0, 2)
    wm = wm.reshape(D_IN, D_EDGE * D_OUT)
    bm = We_b.reshape(D_IN, D_OUT)
    wbig = jnp.concatenate(
        [wm, bm, root, jnp.zeros((D_IN, GP - GW), jnp.float32)], axis=1)

    g = _node_matmul(x, wbig, 10, n // 10)

    zeros = jnp.zeros((n_acc, D_OUT), jnp.float32)
    sc = _make_sc_edges(n_acc, rows_per_tile, n_chunks, ept, last_thresh)
    part = sc(g, src, dst, edge_attr.astype(jnp.float32), zeros)

    return _final(part, g, conv_bias.reshape(1, D_OUT), Wp,
                  bp.reshape(1, D_OUT), 10, n // 10)


# speedup vs baseline: 4.9556x; 1.0137x over previous
import functools

import jax
import jax.numpy as jnp
from jax import lax
from jax.experimental import pallas as pl
from jax.experimental.pallas import tpu as pltpu
from jax.experimental.pallas import tpu_sc as plsc

D_IN = 128
D_EDGE = 16
D_OUT = 8
KX = D_EDGE + 1
GW = KX * D_OUT + D_OUT
GP = 152
CHUNK = 128
NW = 32


def _node_matmul(x, wbig, n_blocks, blk):
    def body(x_ref, w_ref, o_ref):
        o_ref[:, :] = jnp.dot(x_ref[:, :], w_ref[:, :],
                              preferred_element_type=jnp.float32)

    n = x.shape[0]
    return pl.pallas_call(
        body,
        grid=(n_blocks,),
        in_specs=[pl.BlockSpec((blk, D_IN), lambda i: (i, 0)),
                  pl.BlockSpec((D_IN, GP), lambda i: (0, 0))],
        out_specs=pl.BlockSpec((blk, GP), lambda i: (i, 0)),
        out_shape=jax.ShapeDtypeStruct((n, GP), jnp.float32),
    )(x, wbig)


def _make_sc_edges(n_acc, rows_per_tile, n_chunks, ept, last_thresh):
    mesh = plsc.VectorSubcoreMesh(core_axis_name="c", subcore_axis_name="s")
    assert n_chunks >= 3
    nb, tail = divmod(n_chunks, 3)

    scratch = (
        [pltpu.VMEM((CHUNK,), jnp.int32)] * 3
        + [pltpu.VMEM((CHUNK,), jnp.int32)] * 3
        + [pltpu.VMEM((CHUNK, D_EDGE + 1), jnp.float32)] * 3
        + [pltpu.VMEM((CHUNK, GP), jnp.float32)] * 3
        + [pltpu.VMEM((CHUNK, D_OUT), jnp.float32)] * 3
        + [pltpu.VMEM_SHARED((n_acc, D_OUT), jnp.float32)]
        + [pltpu.SemaphoreType.DMA] * 9
    )

    @functools.partial(
        pl.kernel,
        mesh=mesh,
        compiler_params=pltpu.CompilerParams(use_tc_tiling_on_sc=False,
                                             needs_layout_passes=False),
        out_type=jax.ShapeDtypeStruct((2, n_acc, D_OUT), jnp.float32),
        scratch_types=scratch,
    )
    def sc_edges(g_hbm, ei_hbm, attr_hbm, zeros_hbm, out_hbm,
                 *bufs):
        src_b = bufs[0:3]
        dst_b = bufs[3:6]
        attr_b = bufs[6:9]
        rows_b = bufs[9:12]
        msg_b = bufs[12:15]
        acc_s = bufs[15]
        se_idx = bufs[16:19]
        se_rows = bufs[19:22]
        se_sc = bufs[22:25]

        c = lax.axis_index("c")
        s = lax.axis_index("s")
        wid = s * 2 + c
        ebase = wid * ept

        def run(cond, fn):
            if isinstance(cond, bool):
                if cond:
                    fn()
            else:
                pl.when(cond)(fn)

        def chunk_off(ci):
            if isinstance(ci, int):
                return ept - CHUNK if ci == n_chunks - 1 else ci * CHUNK
            return jnp.where(ci == n_chunks - 1, ept - CHUNK, ci * CHUNK)

        def issue_idx(ci, b):
            base = ebase + chunk_off(ci)
            pltpu.async_copy(ei_hbm.at[0, pl.ds(base, CHUNK)], src_b[b],
                             se_idx[b])
            pltpu.async_copy(ei_hbm.at[1, pl.ds(base, CHUNK)], dst_b[b],
                             se_idx[b])
            pltpu.async_copy(attr_hbm.at[pl.ds(base, CHUNK)],
                             attr_b[b].at[:, pl.ds(0, D_EDGE)], se_idx[b])

        def wait_idx(b):
            pltpu.make_async_copy(ei_hbm.at[0, pl.ds(0, CHUNK)], src_b[b],
                                  se_idx[b]).wait()
            pltpu.make_async_copy(ei_hbm.at[1, pl.ds(0, CHUNK)], dst_b[b],
                                  se_idx[b]).wait()
            pltpu.make_async_copy(attr_hbm.at[pl.ds(0, CHUNK)],
                                  attr_b[b].at[:, pl.ds(0, D_EDGE)],
                                  se_idx[b]).wait()

        def issue_rows(b):
            pltpu.async_copy(g_hbm.at[src_b[b]], rows_b[b], se_rows[b])

        def wait_rows(b):
            pltpu.make_async_copy(g_hbm.at[src_b[b]], rows_b[b],
                                  se_rows[b]).wait()

        def issue_scatter(b):
            pltpu.async_copy(msg_b[b], acc_s.at[dst_b[b]], se_sc[b],
                             add=True)

        def wait_scatter(b):
            pltpu.make_async_copy(msg_b[b], acc_s.at[dst_b[b]],
                                  se_sc[b]).wait()

        def compute(ci, b):
            thresh = jnp.where(ci == n_chunks - 1, last_thresh, 0)
            rows_v = rows_b[b]
            attr_v = attr_b[b]
            msg_v = msg_b[b]

            def group_body(g, carry2):
                eidx = lax.iota(jnp.int32, 16) + g * 16
                acc = [jnp.zeros((16,), jnp.float32) for _ in range(D_OUT)]
                for k in range(D_EDGE):
                    a_k = plsc.load_gather(
                        attr_v, [eidx, jnp.full((16,), k, jnp.int32)])
                    for o in range(D_OUT):
                        col = jnp.full((16,), k * D_OUT + o, jnp.int32)
                        vals = plsc.load_gather(rows_v, [eidx, col])
                        acc[o] = acc[o] + a_k * vals
                keep = eidx >= thresh
                for o in range(D_OUT):
                    colb = jnp.full((16,), D_EDGE * D_OUT + o, jnp.int32)
                    bias = plsc.load_gather(rows_v, [eidx, colb])
                    val = jnp.where(keep, acc[o] + bias, 0.0)
                    colo = jnp.full((16,), o, jnp.int32)
                    plsc.store_scatter(msg_v, [eidx, colo], val)
                return carry2

            lax.fori_loop(0, CHUNK // 16, group_body, 0)

        def chunk_step(ci, b):
            wait_rows(b)
            b1 = (b + 1) % 3
            b2 = (b + 2) % 3
            def prefetch_rows():
                wait_idx(b1)
                issue_rows(b1)

            run(ci + 1 < n_chunks, prefetch_rows)
            compute(ci, b)

            def prefetch_idx():
                run(ci >= 1, lambda: wait_scatter(b2))
                issue_idx(ci + 2, b2)

            if isinstance(ci, int):
                if ci + 2 < n_chunks:
                    if ci >= 1:
                        wait_scatter(b2)
                    issue_idx(ci + 2, b2)
            else:
                run(ci + 2 < n_chunks, prefetch_idx)
            issue_scatter(b)

        issue_idx(0, 0)
        issue_idx(1, 1)
        row0 = s * rows_per_tile
        pltpu.sync_copy(zeros_hbm,
                        acc_s.at[pl.ds(row0, rows_per_tile)])
        plsc.subcore_barrier()
        wait_idx(0)
        issue_rows(0)

        def block_body(blk, carry):
            ci0 = blk * 3
            chunk_step(ci0, 0)
            chunk_step(ci0 + 1, 1)
            chunk_step(ci0 + 2, 2)
            return carry

        lax.fori_loop(0, nb, block_body, 0)
        for t in range(tail):
            chunk_step(nb * 3 + t, t)

        for j in range(n_chunks - 3, n_chunks):
            wait_scatter(j % 3)

        plsc.subcore_barrier()
        pltpu.sync_copy(acc_s.at[pl.ds(row0, rows_per_tile)],
                        out_hbm.at[c, pl.ds(row0, rows_per_tile)])

    return sc_edges


def _final(part, g, conv_bias, wp, bp, n_blocks, blk):
    def body(p_ref, g_ref, cb_ref, wp_ref, bp_ref, o_ref):
        aggr = p_ref[0] + p_ref[1]
        pre = aggr + g_ref[:, KX * D_OUT:GW] + cb_ref[:, :]
        h = jnp.maximum(pre, 0.0)
        o_ref[:, :] = (jnp.dot(h, wp_ref[:, :],
                               preferred_element_type=jnp.float32)
                       + bp_ref[:, :])

    n = g.shape[0]
    return pl.pallas_call(
        body,
        grid=(n_blocks,),
        in_specs=[
            pl.BlockSpec((2, blk, D_OUT), lambda i: (0, i, 0)),
            pl.BlockSpec((blk, GP), lambda i: (i, 0)),
            pl.BlockSpec((1, D_OUT), lambda i: (0, 0)),
            pl.BlockSpec((D_OUT, D_OUT), lambda i: (0, 0)),
            pl.BlockSpec((1, D_OUT), lambda i: (0, 0)),
        ],
        out_specs=pl.BlockSpec((blk, D_OUT), lambda i: (i, 0)),
        out_shape=jax.ShapeDtypeStruct((n, D_OUT), jnp.float32),
    )(part, g, conv_bias, wp, bp)


def kernel(x, edge_index, edge_attr, We_w, We_b, root, conv_bias, Wp, bp):
    x = x.astype(jnp.float32)
    n = x.shape[0]
    e = edge_attr.shape[0]
    edge_index = edge_index.astype(jnp.int32)

    assert e % NW == 0 and (e // NW) % 8 == 0 and e // NW >= CHUNK
    ept = e // NW
    n_chunks = -(-ept // CHUNK)
    last_thresh = (CHUNK - ept % CHUNK) % CHUNK
    rows_per_tile = 8 * (-(-(n + 1) // (16 * 8)))
    n_acc = 16 * rows_per_tile

    wm = We_w.reshape(D_EDGE, D_IN, D_OUT).transpose(1, 0, 2)
    wm = wm.reshape(D_IN, D_EDGE * D_OUT)
    bm = We_b.reshape(D_IN, D_OUT)
    wbig = jnp.concatenate(
        [wm, bm, root, jnp.zeros((D_IN, GP - GW), jnp.float32)], axis=1)

    g = _node_matmul(x, wbig, 10, n // 10)

    zeros = jnp.zeros((rows_per_tile, D_OUT), jnp.float32)
    sc = _make_sc_edges(n_acc, rows_per_tile, n_chunks, ept, last_thresh)
    part = sc(g, edge_index, edge_attr.astype(jnp.float32), zeros)

    return _final(part, g, conv_bias.reshape(1, D_OUT), Wp,
                  bp.reshape(1, D_OUT), 10, n // 10)


# speedup vs baseline: 7.1076x; 1.4343x over previous
import functools

import jax
import jax.numpy as jnp
from jax import lax
from jax.experimental import pallas as pl
from jax.experimental.pallas import tpu as pltpu
from jax.experimental.pallas import tpu_sc as plsc

D_IN = 128
D_EDGE = 16
D_OUT = 8
KX = D_EDGE + 1
GW = KX * D_OUT + D_OUT
GP = 152
CHUNK = 128
NW = 32


def _node_matmul(x, wbig, n_blocks, blk):
    def body(x_ref, w_ref, o_ref):
        o_ref[:, :] = jnp.dot(x_ref[:, :], w_ref[:, :],
                              preferred_element_type=jnp.float32)

    n = x.shape[0]
    return pl.pallas_call(
        body,
        grid=(n_blocks,),
        in_specs=[pl.BlockSpec((blk, D_IN), lambda i: (i, 0)),
                  pl.BlockSpec((D_IN, GP), lambda i: (0, 0))],
        out_specs=pl.BlockSpec((blk, GP), lambda i: (i, 0)),
        out_shape=jax.ShapeDtypeStruct((n, GP), jnp.float32),
    )(x, wbig)


def _make_sc_edges(n_acc, rows_per_tile, n_chunks, ept, last_thresh):
    mesh = plsc.VectorSubcoreMesh(core_axis_name="c", subcore_axis_name="s")
    assert n_chunks >= 3
    nb, tail = divmod(n_chunks, 3)

    scratch = (
        [pltpu.VMEM((CHUNK,), jnp.int32)] * 3
        + [pltpu.VMEM((CHUNK,), jnp.int32)] * 3
        + [pltpu.VMEM((CHUNK, D_EDGE + 1), jnp.float32)] * 3
        + [pltpu.VMEM((CHUNK, GP), jnp.float32)] * 3
        + [pltpu.VMEM((CHUNK, D_OUT), jnp.float32)] * 3
        + [pltpu.VMEM_SHARED((n_acc, D_OUT), jnp.float32)]
        + [pltpu.SemaphoreType.DMA] * 9
    )

    @functools.partial(
        pl.kernel,
        mesh=mesh,
        compiler_params=pltpu.CompilerParams(use_tc_tiling_on_sc=False,
                                             needs_layout_passes=False),
        out_type=jax.ShapeDtypeStruct((2, n_acc, D_OUT), jnp.float32),
        scratch_types=scratch,
    )
    def sc_edges(g_hbm, ei_hbm, attr_hbm, zeros_hbm, out_hbm,
                 *bufs):
        src_b = bufs[0:3]
        dst_b = bufs[3:6]
        attr_b = bufs[6:9]
        rows_b = bufs[9:12]
        msg_b = bufs[12:15]
        acc_s = bufs[15]
        se_idx = bufs[16:19]
        se_rows = bufs[19:22]
        se_sc = bufs[22:25]

        c = lax.axis_index("c")
        s = lax.axis_index("s")
        wid = s * 2 + c
        ebase = wid * ept

        def run(cond, fn):
            if isinstance(cond, bool):
                if cond:
                    fn()
            else:
                pl.when(cond)(fn)

        def chunk_off(ci):
            if isinstance(ci, int):
                return ept - CHUNK if ci == n_chunks - 1 else ci * CHUNK
            return jnp.where(ci == n_chunks - 1, ept - CHUNK, ci * CHUNK)

        def issue_idx(ci, b):
            base = ebase + chunk_off(ci)
            pltpu.async_copy(ei_hbm.at[0, pl.ds(base, CHUNK)], src_b[b],
                             se_idx[b])
            pltpu.async_copy(ei_hbm.at[1, pl.ds(base, CHUNK)], dst_b[b],
                             se_idx[b])
            pltpu.async_copy(attr_hbm.at[pl.ds(base, CHUNK)],
                             attr_b[b].at[:, pl.ds(0, D_EDGE)], se_idx[b])

        def wait_idx(b):
            pltpu.make_async_copy(ei_hbm.at[0, pl.ds(0, CHUNK)], src_b[b],
                                  se_idx[b]).wait()
            pltpu.make_async_copy(ei_hbm.at[1, pl.ds(0, CHUNK)], dst_b[b],
                                  se_idx[b]).wait()
            pltpu.make_async_copy(attr_hbm.at[pl.ds(0, CHUNK)],
                                  attr_b[b].at[:, pl.ds(0, D_EDGE)],
                                  se_idx[b]).wait()

        def issue_rows(b):
            pltpu.async_copy(g_hbm.at[src_b[b]], rows_b[b], se_rows[b])

        def wait_rows(b):
            pltpu.make_async_copy(g_hbm.at[src_b[b]], rows_b[b],
                                  se_rows[b]).wait()

        def issue_scatter(b):
            pltpu.async_copy(msg_b[b], acc_s.at[dst_b[b]], se_sc[b],
                             add=True)

        def wait_scatter(b):
            pltpu.make_async_copy(msg_b[b], acc_s.at[dst_b[b]],
                                  se_sc[b]).wait()

        def compute(ci, b):
            thresh = jnp.where(ci == n_chunks - 1, last_thresh, 0)
            rows_v = rows_b[b]
            attr_v = attr_b[b]
            msg_v = msg_b[b]

            @functools.partial(plsc.parallel_loop, 0, CHUNK // 16,
                               unroll=2)
            def group_body(g):
                eidx = lax.iota(jnp.int32, 16) + g * 16
                acc = [jnp.zeros((16,), jnp.float32) for _ in range(D_OUT)]
                for k in range(D_EDGE):
                    a_k = plsc.load_gather(
                        attr_v, [eidx, jnp.full((16,), k, jnp.int32)])
                    for o in range(D_OUT):
                        col = jnp.full((16,), k * D_OUT + o, jnp.int32)
                        vals = plsc.load_gather(rows_v, [eidx, col])
                        acc[o] = acc[o] + a_k * vals
                keep = eidx >= thresh
                for o in range(D_OUT):
                    colb = jnp.full((16,), D_EDGE * D_OUT + o, jnp.int32)
                    bias = plsc.load_gather(rows_v, [eidx, colb])
                    val = jnp.where(keep, acc[o] + bias, 0.0)
                    colo = jnp.full((16,), o, jnp.int32)
                    plsc.store_scatter(msg_v, [eidx, colo], val)

        def chunk_step(ci, b):
            wait_rows(b)
            b1 = (b + 1) % 3
            b2 = (b + 2) % 3
            def prefetch_rows():
                wait_idx(b1)
                issue_rows(b1)

            run(ci + 1 < n_chunks, prefetch_rows)
            compute(ci, b)

            def prefetch_idx():
                run(ci >= 1, lambda: wait_scatter(b2))
                issue_idx(ci + 2, b2)

            if isinstance(ci, int):
                if ci + 2 < n_chunks:
                    if ci >= 1:
                        wait_scatter(b2)
                    issue_idx(ci + 2, b2)
            else:
                run(ci + 2 < n_chunks, prefetch_idx)
            issue_scatter(b)

        issue_idx(0, 0)
        issue_idx(1, 1)
        row0 = s * rows_per_tile
        pltpu.sync_copy(zeros_hbm,
                        acc_s.at[pl.ds(row0, rows_per_tile)])
        plsc.subcore_barrier()
        wait_idx(0)
        issue_rows(0)

        def block_body(blk, carry):
            ci0 = blk * 3
            chunk_step(ci0, 0)
            chunk_step(ci0 + 1, 1)
            chunk_step(ci0 + 2, 2)
            return carry

        lax.fori_loop(0, nb, block_body, 0)
        for t in range(tail):
            chunk_step(nb * 3 + t, t)

        for j in range(n_chunks - 3, n_chunks):
            wait_scatter(j % 3)

        plsc.subcore_barrier()
        pltpu.sync_copy(acc_s.at[pl.ds(row0, rows_per_tile)],
                        out_hbm.at[c, pl.ds(row0, rows_per_tile)])

    return sc_edges


def _final(part, g, conv_bias, wp, bp, n_blocks, blk):
    def body(p_ref, g_ref, cb_ref, wp_ref, bp_ref, o_ref):
        aggr = p_ref[0] + p_ref[1]
        pre = aggr + g_ref[:, KX * D_OUT:GW] + cb_ref[:, :]
        h = jnp.maximum(pre, 0.0)
        o_ref[:, :] = (jnp.dot(h, wp_ref[:, :],
                               preferred_element_type=jnp.float32)
                       + bp_ref[:, :])

    n = g.shape[0]
    return pl.pallas_call(
        body,
        grid=(n_blocks,),
        in_specs=[
            pl.BlockSpec((2, blk, D_OUT), lambda i: (0, i, 0)),
            pl.BlockSpec((blk, GP), lambda i: (i, 0)),
            pl.BlockSpec((1, D_OUT), lambda i: (0, 0)),
            pl.BlockSpec((D_OUT, D_OUT), lambda i: (0, 0)),
            pl.BlockSpec((1, D_OUT), lambda i: (0, 0)),
        ],
        out_specs=pl.BlockSpec((blk, D_OUT), lambda i: (i, 0)),
        out_shape=jax.ShapeDtypeStruct((n, D_OUT), jnp.float32),
    )(part, g, conv_bias, wp, bp)


def kernel(x, edge_index, edge_attr, We_w, We_b, root, conv_bias, Wp, bp):
    x = x.astype(jnp.float32)
    n = x.shape[0]
    e = edge_attr.shape[0]
    edge_index = edge_index.astype(jnp.int32)

    assert e % NW == 0 and (e // NW) % 8 == 0 and e // NW >= CHUNK
    ept = e // NW
    n_chunks = -(-ept // CHUNK)
    last_thresh = (CHUNK - ept % CHUNK) % CHUNK
    rows_per_tile = 8 * (-(-(n + 1) // (16 * 8)))
    n_acc = 16 * rows_per_tile

    wm = We_w.reshape(D_EDGE, D_IN, D_OUT).transpose(1, 0, 2)
    wm = wm.reshape(D_IN, D_EDGE * D_OUT)
    bm = We_b.reshape(D_IN, D_OUT)
    wbig = jnp.concatenate(
        [wm, bm, root, jnp.zeros((D_IN, GP - GW), jnp.float32)], axis=1)

    g = _node_matmul(x, wbig, 10, n // 10)

    zeros = jnp.zeros((rows_per_tile, D_OUT), jnp.float32)
    sc = _make_sc_edges(n_acc, rows_per_tile, n_chunks, ept, last_thresh)
    part = sc(g, edge_index, edge_attr.astype(jnp.float32), zeros)

    return _final(part, g, conv_bias.reshape(1, D_OUT), Wp,
                  bp.reshape(1, D_OUT), 10, n // 10)
